# Initial kernel scaffold; baseline (speedup 1.0000x reference)
import jax
import jax.numpy as jnp
from jax.experimental import pallas as pl


def kernel(x, patch_idx, group_idx, embeddings, W, b):
    raise NotImplementedError("write your pallas kernel here")


# speedup vs baseline: 12.0477x; 12.0477x over previous
import functools

import jax
import jax.numpy as jnp
from jax import lax
from jax.experimental import pallas as pl
from jax.experimental.pallas import tpu as pltpu
from jax.experimental.pallas import tpu_sc as plsc

_NC = 2
_NS = 16
_NW = _NC * _NS

_CHUNK = 128


def _sc_gather(table, idx3):
    nw, j_steps, chunk = idx3.shape
    rows_out = nw * j_steps * chunk
    feat = table.shape[1]
    mesh = plsc.VectorSubcoreMesh(core_axis_name="c", subcore_axis_name="s")

    @functools.partial(
        pl.kernel,
        mesh=mesh,
        out_type=jax.ShapeDtypeStruct((rows_out, feat), jnp.float32),
        scratch_types=[
            pltpu.VMEM((j_steps, chunk), jnp.int32),
            pltpu.VMEM((chunk, feat), jnp.float32),
            pltpu.SemaphoreType.DMA,
        ],
    )
    def gather_k(table_hbm, idx_hbm, out_hbm, idx_v, rows_v, sem):
        wid = lax.axis_index("s") * _NC + lax.axis_index("c")
        pltpu.sync_copy(idx_hbm.at[wid], idx_v)
        base = wid * (j_steps * chunk)

        def body(j, carry):
            pltpu.async_copy(table_hbm.at[idx_v.at[j]], rows_v, sem).wait()
            pltpu.sync_copy(rows_v, out_hbm.at[pl.ds(base + j * chunk, chunk)])
            return carry

        lax.fori_loop(0, j_steps, body, 0)

    return gather_k(table, idx3)


def _film_body(e_ref, x_ref, w_ref, b_ref, out_ref):
    feat = x_ref.shape[-1]
    h = jnp.dot(e_ref[...], w_ref[...], preferred_element_type=jnp.float32)
    h = h + b_ref[...]
    out_ref[...] = x_ref[...] * h[:, :feat] + h[:, feat:]


def kernel(x, patch_idx, group_idx, embeddings, W, b):
    batch, patch, feat = x.shape
    n_groups, n_nodes, _ = embeddings.shape
    rows = batch * patch

    table = embeddings.reshape(n_groups * n_nodes, feat)
    flat_idx = (group_idx.astype(jnp.int32)[:, None] * n_nodes
                + patch_idx.astype(jnp.int32))
    j_steps = rows // (_NW * _CHUNK)
    idx3 = flat_idx.reshape(_NW, j_steps, _CHUNK)

    e = _sc_gather(table, idx3)

    blk = 2048
    out = pl.pallas_call(
        _film_body,
        grid=(rows // blk,),
        in_specs=[
            pl.BlockSpec((blk, feat), lambda i: (i, 0)),
            pl.BlockSpec((blk, feat), lambda i: (i, 0)),
            pl.BlockSpec((feat, 2 * feat), lambda i: (0, 0)),
            pl.BlockSpec((1, 2 * feat), lambda i: (0, 0)),
        ],
        out_specs=pl.BlockSpec((blk, feat), lambda i: (i, 0)),
        out_shape=jax.ShapeDtypeStruct((rows, feat), jnp.float32),
    )(e, x.reshape(rows, feat), W, b.reshape(1, 2 * feat))
    return out.reshape(batch, patch, feat)


# speedup vs baseline: 12.8198x; 1.0641x over previous
import functools

import jax
import jax.numpy as jnp
from jax import lax
from jax.experimental import pallas as pl
from jax.experimental.pallas import tpu as pltpu
from jax.experimental.pallas import tpu_sc as plsc

_NC = 2
_NS = 16
_NW = _NC * _NS

_CHUNK = 128


def _sc_gather(table, idx3):
    nw, j_steps, chunk = idx3.shape
    rows_out = nw * j_steps * chunk
    feat = table.shape[1]
    mesh = plsc.VectorSubcoreMesh(core_axis_name="c", subcore_axis_name="s")

    @functools.partial(
        pl.kernel,
        mesh=mesh,
        out_type=jax.ShapeDtypeStruct((rows_out, feat), jnp.float32),
        scratch_types=[
            pltpu.VMEM((j_steps, chunk), jnp.int32),
            pltpu.VMEM((chunk, feat), jnp.float32),
            pltpu.SemaphoreType.DMA,
        ],
    )
    def gather_k(table_hbm, idx_hbm, out_hbm, idx_v, rows_v, sem):
        wid = lax.axis_index("s") * _NC + lax.axis_index("c")
        pltpu.sync_copy(idx_hbm.at[wid], idx_v)
        base = wid * (j_steps * chunk)

        def body(j, carry):
            pltpu.async_copy(table_hbm.at[idx_v.at[j]], rows_v, sem).wait()
            pltpu.sync_copy(rows_v, out_hbm.at[pl.ds(base + j * chunk, chunk)])
            return carry

        lax.fori_loop(0, j_steps, body, 0)

    return gather_k(table, idx3)


def _film_body(e_ref, x_ref, w_ref, b_ref, out_ref):
    feat = x_ref.shape[-1]
    h = jnp.dot(e_ref[...], w_ref[...], preferred_element_type=jnp.float32)
    h = h + b_ref[...]
    out_ref[...] = x_ref[...] * h[:, :feat] + h[:, feat:]


def _film_body_chained(e_ref, x_ref, w_ref, b_ref, buf_ref, out_ref):
    del buf_ref
    _film_body(e_ref, x_ref, w_ref, b_ref, out_ref)


_K = 4
_BLK = 2048


def _film_chunk(e_k, x2, W, b2, buf, k, rows, feat):
    chunk_rows = e_k.shape[0]
    nb = chunk_rows // _BLK
    e_spec = pl.BlockSpec((_BLK, feat), lambda i: (i, 0))
    x_spec = pl.BlockSpec((_BLK, feat), lambda i: (k * nb + i, 0))
    w_spec = pl.BlockSpec((feat, 2 * feat), lambda i: (0, 0))
    b_spec = pl.BlockSpec((1, 2 * feat), lambda i: (0, 0))
    out_spec = pl.BlockSpec((_BLK, feat), lambda i: (k * nb + i, 0))
    out_shape = jax.ShapeDtypeStruct((rows, feat), jnp.float32)
    if buf is None:
        return pl.pallas_call(
            _film_body,
            grid=(nb,),
            in_specs=[e_spec, x_spec, w_spec, b_spec],
            out_specs=out_spec,
            out_shape=out_shape,
        )(e_k, x2, W, b2)
    buf_spec = pl.BlockSpec((8, feat), lambda i: (0, 0))
    return pl.pallas_call(
        _film_body_chained,
        grid=(nb,),
        in_specs=[e_spec, x_spec, w_spec, b_spec, buf_spec],
        out_specs=out_spec,
        out_shape=out_shape,
        input_output_aliases={4: 0},
    )(e_k, x2, W, b2, buf)


def kernel(x, patch_idx, group_idx, embeddings, W, b):
    batch, patch, feat = x.shape
    n_groups, n_nodes, _ = embeddings.shape
    rows = batch * patch

    table = embeddings.reshape(n_groups * n_nodes, feat)
    flat_idx = (group_idx.astype(jnp.int32)[:, None] * n_nodes
                + patch_idx.astype(jnp.int32))
    j_steps = rows // (_K * _NW * _CHUNK)
    idx4 = flat_idx.reshape(_K, _NW, j_steps, _CHUNK)

    e_chunks = [_sc_gather(table, idx4[k]) for k in range(_K)]

    x2 = x.reshape(rows, feat)
    b2 = b.reshape(1, 2 * feat)
    buf = None
    for k in range(_K):
        buf = _film_chunk(e_chunks[k], x2, W, b2, buf, k, rows, feat)
    return buf.reshape(batch, patch, feat)


# speedup vs baseline: 14.0961x; 1.0996x over previous
import functools

import jax
import jax.numpy as jnp
from jax import lax
from jax.experimental import pallas as pl
from jax.experimental.pallas import tpu as pltpu
from jax.experimental.pallas import tpu_sc as plsc

_NC = 2
_NS = 16
_NW = _NC * _NS

_CHUNK = 128


def _sc_gather(table, idx3):
    nw, j_steps, chunk = idx3.shape
    rows_out = nw * j_steps * chunk
    feat = table.shape[1]
    mesh = plsc.VectorSubcoreMesh(core_axis_name="c", subcore_axis_name="s")

    @functools.partial(
        pl.kernel,
        mesh=mesh,
        out_type=jax.ShapeDtypeStruct((rows_out, feat), jnp.float32),
        scratch_types=(
            [pltpu.VMEM((j_steps, chunk), jnp.int32),
             pltpu.VMEM((j_steps * chunk, feat), jnp.float32)]
            + [pltpu.SemaphoreType.DMA] * j_steps
            + [pltpu.SemaphoreType.DMA]
        ),
    )
    def gather_k(table_hbm, idx_hbm, out_hbm, idx_v, rows_v, *sems):
        gsems, ssem = sems[:j_steps], sems[j_steps]
        wid = lax.axis_index("s") * _NC + lax.axis_index("c")
        pltpu.sync_copy(idx_hbm.at[wid], idx_v)
        base = wid * (j_steps * chunk)
        gathers = [
            pltpu.async_copy(
                table_hbm.at[idx_v.at[j]],
                rows_v.at[pl.ds(j * chunk, chunk)],
                gsems[j],
            )
            for j in range(j_steps)
        ]
        scatters = []
        for j in range(j_steps):
            gathers[j].wait()
            scatters.append(
                pltpu.async_copy(
                    rows_v.at[pl.ds(j * chunk, chunk)],
                    out_hbm.at[pl.ds(base + j * chunk, chunk)],
                    ssem,
                )
            )
        for s in scatters:
            s.wait()

    return gather_k(table, idx3)


def _film_body(e_ref, x_ref, w_ref, b_ref, out_ref):
    feat = x_ref.shape[-1]
    h = jnp.dot(e_ref[...], w_ref[...], preferred_element_type=jnp.float32)
    h = h + b_ref[...]
    out_ref[...] = x_ref[...] * h[:, :feat] + h[:, feat:]


def _film_body_chained(e_ref, x_ref, w_ref, b_ref, buf_ref, out_ref):
    del buf_ref
    _film_body(e_ref, x_ref, w_ref, b_ref, out_ref)


_K = 4
_BLK = 2048


def _film_chunk(e_k, x2, W, b2, buf, k, rows, feat):
    chunk_rows = e_k.shape[0]
    nb = chunk_rows // _BLK
    e_spec = pl.BlockSpec((_BLK, feat), lambda i: (i, 0))
    x_spec = pl.BlockSpec((_BLK, feat), lambda i: (k * nb + i, 0))
    w_spec = pl.BlockSpec((feat, 2 * feat), lambda i: (0, 0))
    b_spec = pl.BlockSpec((1, 2 * feat), lambda i: (0, 0))
    out_spec = pl.BlockSpec((_BLK, feat), lambda i: (k * nb + i, 0))
    out_shape = jax.ShapeDtypeStruct((rows, feat), jnp.float32)
    if buf is None:
        return pl.pallas_call(
            _film_body,
            grid=(nb,),
            in_specs=[e_spec, x_spec, w_spec, b_spec],
            out_specs=out_spec,
            out_shape=out_shape,
        )(e_k, x2, W, b2)
    buf_spec = pl.BlockSpec((8, feat), lambda i: (0, 0))
    return pl.pallas_call(
        _film_body_chained,
        grid=(nb,),
        in_specs=[e_spec, x_spec, w_spec, b_spec, buf_spec],
        out_specs=out_spec,
        out_shape=out_shape,
        input_output_aliases={4: 0},
    )(e_k, x2, W, b2, buf)


def kernel(x, patch_idx, group_idx, embeddings, W, b):
    batch, patch, feat = x.shape
    n_groups, n_nodes, _ = embeddings.shape
    rows = batch * patch

    table = embeddings.reshape(n_groups * n_nodes, feat)
    flat_idx = (group_idx.astype(jnp.int32)[:, None] * n_nodes
                + patch_idx.astype(jnp.int32))
    j_steps = rows // (_K * _NW * _CHUNK)
    idx4 = flat_idx.reshape(_K, _NW, j_steps, _CHUNK)

    e_chunks = [_sc_gather(table, idx4[k]) for k in range(_K)]

    x2 = x.reshape(rows, feat)
    b2 = b.reshape(1, 2 * feat)
    buf = None
    for k in range(_K):
        buf = _film_chunk(e_chunks[k], x2, W, b2, buf, k, rows, feat)
    return buf.reshape(batch, patch, feat)
